# baseline (speedup 1.0000x reference)
import functools

import jax
import jax.numpy as jnp
from jax import lax
from jax.experimental import pallas as pl
from jax.experimental.pallas import tpu as pltpu
from jax.experimental.pallas import tpu_sc as plsc

_N = 10000
_E = 160000
_IN = 256
_HID = 512
_OUT = 256

_NC = 2
_NS = 16
_CHUNK = 128
_EPAD = 163840
_NCHUNKS = _EPAD // _CHUNK
_CPT = _NCHUNKS // _NS
_ZROWS = 632
_ACC_ROWS = _NS * _ZROWS


def _fill_rows(ref, nrows, val):
    v = jnp.full((16,), val, jnp.float32)
    ncol = ref.shape[1]

    def body(i, carry):
        for j in range(ncol // 16):
            ref[i, pl.ds(j * 16, 16)] = v
        return carry

    lax.fori_loop(0, nrows, body, 0)


def _zero_slice(acc, zbuf, zbase):
    for k in range(_ZROWS // _CHUNK):
        pltpu.sync_copy(zbuf, acc.at[pl.ds(zbase + k * _CHUNK, _CHUNK)])
    rem = _ZROWS % _CHUNK
    if rem:
        pltpu.sync_copy(zbuf.at[pl.ds(0, rem)],
                        acc.at[pl.ds(zbase + (_ZROWS // _CHUNK) * _CHUNK, rem)])


_W = 8
_NWIN = _CPT // _W
_QC = 64


def _seg_body(table, src2, dst2, out_agg, src_w, dst_w, buf0, buf1, acc,
              sem0, sem1):
    c = lax.axis_index("c")
    s = lax.axis_index("s")
    zbase = s * _ZROWS

    bufs = (buf0, buf1)
    sems = (sem0, sem1)

    _fill_rows(buf0, _CHUNK, 0.0)
    _zero_slice(acc, buf0, zbase)

    plsc.subcore_barrier()

    def window(w, carry):
        base = s * _CPT + w * _W
        pltpu.sync_copy(src2.at[c, pl.ds(base, _W)], src_w)
        pltpu.sync_copy(dst2.at[pl.ds(base, _W)], dst_w)
        cps = [None, None]
        cps[0] = pltpu.async_copy(table.at[src_w.at[0]], buf0, sem0)
        for j in range(_W):
            nxt = (j + 1) % 2
            if j + 1 < _W:
                cps[nxt] = pltpu.async_copy(table.at[src_w.at[j + 1]],
                                            bufs[nxt], sems[nxt])
            cps[j % 2].wait()
            pltpu.sync_copy(bufs[j % 2], acc.at[dst_w.at[j]], add=True)
        return carry

    lax.fori_loop(0, _NWIN, window, 0)

    plsc.subcore_barrier()

    pltpu.sync_copy(acc.at[pl.ds(zbase, _ZROWS)],
                    out_agg.at[c, pl.ds(zbase, _ZROWS)])


_DEG_CPT = _NCHUNKS // (_NC * _NS)


def _deg_body(dst2, out_deg, dst_v, ones_v, deg_acc):
    c = lax.axis_index("c")
    s = lax.axis_index("s")

    base = c * (_NS * _DEG_CPT) + s * _DEG_CPT
    pltpu.sync_copy(dst2.at[pl.ds(base, _DEG_CPT)], dst_v)

    _fill_rows(ones_v, _CHUNK, 0.0)
    _zero_slice(deg_acc, ones_v, s * _ZROWS)
    _fill_rows(ones_v, _CHUNK, 1.0)

    plsc.subcore_barrier()

    def chunk(j, carry):
        pltpu.sync_copy(ones_v, deg_acc.at[dst_v.at[j]], add=True)
        return carry

    lax.fori_loop(0, _DEG_CPT, chunk, 0)

    plsc.subcore_barrier()

    obase = s * _ZROWS
    pltpu.sync_copy(deg_acc.at[pl.ds(obase, _ZROWS)],
                    out_deg.at[c, pl.ds(obase, _ZROWS)])


def _make_seg():
    mesh = plsc.VectorSubcoreMesh(core_axis_name="c", subcore_axis_name="s",
                                  num_cores=_NC, num_subcores=_NS)
    scratch = [
        pltpu.VMEM((_W, _CHUNK), jnp.int32),
        pltpu.VMEM((_W, _CHUNK), jnp.int32),
        pltpu.VMEM((_CHUNK, 128), jnp.float32),
        pltpu.VMEM((_CHUNK, 128), jnp.float32),
        pltpu.VMEM_SHARED((_ACC_ROWS, 128), jnp.float32),
        pltpu.SemaphoreType.DMA,
        pltpu.SemaphoreType.DMA,
    ]
    return pl.kernel(
        _seg_body,
        out_type=(jax.ShapeDtypeStruct((_NC, _ACC_ROWS, 128), jnp.float32),),
        mesh=mesh, scratch_types=scratch)


def _make_deg():
    mesh = plsc.VectorSubcoreMesh(core_axis_name="c", subcore_axis_name="s",
                                  num_cores=_NC, num_subcores=_NS)
    scratch = [
        pltpu.VMEM((_DEG_CPT, _CHUNK), jnp.int32),
        pltpu.VMEM((_CHUNK, 128), jnp.float32),
        pltpu.VMEM_SHARED((_ACC_ROWS, 128), jnp.float32),
    ]
    return pl.kernel(
        _deg_body,
        out_type=(jax.ShapeDtypeStruct((_NC, _ACC_ROWS, 128), jnp.float32),),
        mesh=mesh, scratch_types=scratch)


_BN = 400


def _tc1_body(agg_ref, x_ref, deg_ref, w1l_ref, w1r_ref, b1_ref,
              w2l_ref, w2r_ref, b2_ref, p4_ref, r_ref):
    deg = deg_ref[0, :, 0:1] + deg_ref[1, :, 0:1]
    inv = 1.0 / jnp.maximum(deg, 1.0)
    bf = jnp.bfloat16
    ssum = jnp.dot(agg_ref[0].astype(bf), w1l_ref[:128, :].astype(bf),
                   preferred_element_type=jnp.float32)
    ssum = ssum + jnp.dot(agg_ref[1].astype(bf), w1l_ref[128:, :].astype(bf),
                          preferred_element_type=jnp.float32)
    h = ssum * inv + jnp.dot(x_ref[...].astype(bf), w1r_ref[...].astype(bf),
                             preferred_element_type=jnp.float32) + b1_ref[...]
    h = jnp.maximum(h, 0.0).astype(bf)
    p = jnp.dot(h, w2l_ref[...].astype(bf),
                preferred_element_type=jnp.float32)
    p4_ref[0] = p[:, :128]
    p4_ref[1] = p[:, 128:]
    r_ref[...] = jnp.dot(h, w2r_ref[...].astype(bf),
                         preferred_element_type=jnp.float32) + b2_ref[...]


def _tc2_body(agg_ref, deg_ref, r_ref, out_ref):
    deg = deg_ref[0, :, 0:1] + deg_ref[1, :, 0:1]
    inv = 1.0 / jnp.maximum(deg, 1.0)
    out_ref[...] = jnp.concatenate(
        [agg_ref[0], agg_ref[1]], axis=1) * inv + r_ref[...]


def _tc1(agg1, x, deg, W1_l, W1_r, b1, W2_l, W2_r, b2):
    grid = (_N // _BN,)
    return pl.pallas_call(
        _tc1_body,
        grid=grid,
        in_specs=[
            pl.BlockSpec((_NC, _BN, 128), lambda i: (0, i, 0)),
            pl.BlockSpec((_BN, _IN), lambda i: (i, 0)),
            pl.BlockSpec((_NC, _BN, 128), lambda i: (0, i, 0)),
            pl.BlockSpec((_IN, _HID), lambda i: (0, 0)),
            pl.BlockSpec((_IN, _HID), lambda i: (0, 0)),
            pl.BlockSpec((1, _HID), lambda i: (0, 0)),
            pl.BlockSpec((_HID, _OUT), lambda i: (0, 0)),
            pl.BlockSpec((_HID, _OUT), lambda i: (0, 0)),
            pl.BlockSpec((1, _OUT), lambda i: (0, 0)),
        ],
        out_specs=[
            pl.BlockSpec((_NC, _BN, 128), lambda i: (0, i, 0)),
            pl.BlockSpec((_BN, _OUT), lambda i: (i, 0)),
        ],
        out_shape=[
            jax.ShapeDtypeStruct((_NC, _N, 128), jnp.float32),
            jax.ShapeDtypeStruct((_N, _OUT), jnp.float32),
        ],
    )(agg1, x, deg, W1_l, W1_r, b1, W2_l, W2_r, b2)


def _tc2(agg2, deg, r):
    grid = (_N // _BN,)
    return pl.pallas_call(
        _tc2_body,
        grid=grid,
        in_specs=[
            pl.BlockSpec((_NC, _BN, 128), lambda i: (0, i, 0)),
            pl.BlockSpec((_NC, _BN, 128), lambda i: (0, i, 0)),
            pl.BlockSpec((_BN, _OUT), lambda i: (i, 0)),
        ],
        out_specs=pl.BlockSpec((_BN, _OUT), lambda i: (i, 0)),
        out_shape=jax.ShapeDtypeStruct((_N, _OUT), jnp.float32),
    )(agg2, deg, r)


def kernel(x, edge_index, W1_l, W1_r, b1, W2_l, W2_r, b2):
    src = edge_index[0]
    dst = edge_index[1]
    pad = _EPAD - _E
    iota = jnp.arange(pad, dtype=jnp.int32)
    src_p = jnp.concatenate([src, iota * 37 % _N])
    dst_p = jnp.concatenate([dst, _N + iota % (_ACC_ROWS - _N)])
    src2 = jnp.stack([src_p, src_p + _N]).reshape(_NC, _NCHUNKS, _CHUNK)
    dst2 = dst_p.reshape(_NCHUNKS, _CHUNK)
    x2 = jnp.concatenate([x[:, :128], x[:, 128:]], axis=0)

    deg, = _make_deg()(dst2)
    agg1, = _make_seg()(x2, src2, dst2)
    p2, r = _tc1(agg1, x, deg, W1_l, W1_r, b1.reshape(1, _HID),
                 W2_l, W2_r, b2.reshape(1, _OUT))
    agg2, = _make_seg()(p2.reshape(_NC * _N, 128), src2, dst2)
    return _tc2(agg2, deg, r)


# speedup vs baseline: 1.0083x; 1.0083x over previous
import functools

import jax
import jax.numpy as jnp
from jax import lax
from jax.experimental import pallas as pl
from jax.experimental.pallas import tpu as pltpu
from jax.experimental.pallas import tpu_sc as plsc

_N = 10000
_E = 160000
_IN = 256
_HID = 512
_OUT = 256

_NC = 2
_NS = 16
_CHUNK = 128
_EPAD = 163840
_NCHUNKS = _EPAD // _CHUNK
_CPT = _NCHUNKS // _NS
_ZROWS = 632
_ACC_ROWS = _NS * _ZROWS


def _fill_rows(ref, nrows, val):
    v = jnp.full((16,), val, jnp.float32)
    ncol = ref.shape[1]

    def body(i, carry):
        for j in range(ncol // 16):
            ref[i, pl.ds(j * 16, 16)] = v
        return carry

    lax.fori_loop(0, nrows, body, 0)


def _zero_slice(acc, zbuf, zbase):
    for k in range(_ZROWS // _CHUNK):
        pltpu.sync_copy(zbuf, acc.at[pl.ds(zbase + k * _CHUNK, _CHUNK)])
    rem = _ZROWS % _CHUNK
    if rem:
        pltpu.sync_copy(zbuf.at[pl.ds(0, rem)],
                        acc.at[pl.ds(zbase + (_ZROWS // _CHUNK) * _CHUNK, rem)])


_W = 8
_NWIN = _CPT // _W
_QC = 64


def _seg_body(compute_deg, table, src2, dst2, *rest):
    if compute_deg:
        (out_agg, out_deg, src_w, dst_w, buf0, buf1, acc, sem0, sem1) = rest
    else:
        (out_agg, src_w, dst_w, buf0, buf1, acc, sem0, sem1) = rest
    c = lax.axis_index("c")
    s = lax.axis_index("s")
    zbase = s * _ZROWS

    bufs = (buf0, buf1)
    sems = (sem0, sem1)

    _fill_rows(buf0, _CHUNK, 0.0)
    _zero_slice(acc, buf0, zbase)

    if compute_deg:
        _fill_rows(buf1, _CHUNK, 1.0)
        plsc.subcore_barrier()

        def dwin(w, carry):
            base = c * (_NS * _DEG_CPT) + s * _DEG_CPT + w * _W
            pltpu.sync_copy(dst2.at[pl.ds(base, _W)], dst_w)
            for j in range(_W):
                pltpu.sync_copy(buf1, acc.at[dst_w.at[j]], add=True)
            return carry

        lax.fori_loop(0, _DEG_CPT // _W, dwin, 0)

        plsc.subcore_barrier()
        pltpu.sync_copy(acc.at[pl.ds(zbase, _ZROWS)],
                        out_deg.at[c, pl.ds(zbase, _ZROWS)])
        _zero_slice(acc, buf0, zbase)

    plsc.subcore_barrier()

    def window(w, carry):
        base = s * _CPT + w * _W
        pltpu.sync_copy(src2.at[c, pl.ds(base, _W)], src_w)
        pltpu.sync_copy(dst2.at[pl.ds(base, _W)], dst_w)
        cps = [None, None]
        cps[0] = pltpu.async_copy(table.at[src_w.at[0]], buf0, sem0)
        for j in range(_W):
            nxt = (j + 1) % 2
            if j + 1 < _W:
                cps[nxt] = pltpu.async_copy(table.at[src_w.at[j + 1]],
                                            bufs[nxt], sems[nxt])
            cps[j % 2].wait()
            pltpu.sync_copy(bufs[j % 2], acc.at[dst_w.at[j]], add=True)
        return carry

    lax.fori_loop(0, _NWIN, window, 0)

    plsc.subcore_barrier()

    pltpu.sync_copy(acc.at[pl.ds(zbase, _ZROWS)],
                    out_agg.at[c, pl.ds(zbase, _ZROWS)])


_DEG_CPT = _NCHUNKS // (_NC * _NS)


def _make_seg(compute_deg):
    mesh = plsc.VectorSubcoreMesh(core_axis_name="c", subcore_axis_name="s",
                                  num_cores=_NC, num_subcores=_NS)
    out_type = [jax.ShapeDtypeStruct((_NC, _ACC_ROWS, 128), jnp.float32)]
    if compute_deg:
        out_type.append(
            jax.ShapeDtypeStruct((_NC, _ACC_ROWS, 128), jnp.float32))
    scratch = [
        pltpu.VMEM((_W, _CHUNK), jnp.int32),
        pltpu.VMEM((_W, _CHUNK), jnp.int32),
        pltpu.VMEM((_CHUNK, 128), jnp.float32),
        pltpu.VMEM((_CHUNK, 128), jnp.float32),
        pltpu.VMEM_SHARED((_ACC_ROWS, 128), jnp.float32),
        pltpu.SemaphoreType.DMA,
        pltpu.SemaphoreType.DMA,
    ]
    return pl.kernel(
        functools.partial(_seg_body, compute_deg),
        out_type=tuple(out_type), mesh=mesh, scratch_types=scratch)


_BN = 400


def _tc1_body(agg_ref, x_ref, deg_ref, w1l_ref, w1r_ref, b1_ref,
              w2l_ref, w2r_ref, b2_ref, p4_ref, r_ref):
    deg = deg_ref[0, :, 0:1] + deg_ref[1, :, 0:1]
    inv = 1.0 / jnp.maximum(deg, 1.0)
    ssum = jnp.dot(agg_ref[0], w1l_ref[:128, :],
                   preferred_element_type=jnp.float32)
    ssum = ssum + jnp.dot(agg_ref[1], w1l_ref[128:, :],
                          preferred_element_type=jnp.float32)
    h = ssum * inv + jnp.dot(x_ref[...], w1r_ref[...],
                             preferred_element_type=jnp.float32) + b1_ref[...]
    h = jnp.maximum(h, 0.0)
    p = jnp.dot(h, w2l_ref[...], preferred_element_type=jnp.float32)
    p4_ref[0] = p[:, :128]
    p4_ref[1] = p[:, 128:]
    r_ref[...] = jnp.dot(h, w2r_ref[...],
                         preferred_element_type=jnp.float32) + b2_ref[...]


def _tc2_body(agg_ref, deg_ref, r_ref, out_ref):
    deg = deg_ref[0, :, 0:1] + deg_ref[1, :, 0:1]
    inv = 1.0 / jnp.maximum(deg, 1.0)
    out_ref[...] = jnp.concatenate(
        [agg_ref[0], agg_ref[1]], axis=1) * inv + r_ref[...]


def _tc1(agg1, x, deg, W1_l, W1_r, b1, W2_l, W2_r, b2):
    grid = (_N // _BN,)
    return pl.pallas_call(
        _tc1_body,
        grid=grid,
        in_specs=[
            pl.BlockSpec((_NC, _BN, 128), lambda i: (0, i, 0)),
            pl.BlockSpec((_BN, _IN), lambda i: (i, 0)),
            pl.BlockSpec((_NC, _BN, 128), lambda i: (0, i, 0)),
            pl.BlockSpec((_IN, _HID), lambda i: (0, 0)),
            pl.BlockSpec((_IN, _HID), lambda i: (0, 0)),
            pl.BlockSpec((1, _HID), lambda i: (0, 0)),
            pl.BlockSpec((_HID, _OUT), lambda i: (0, 0)),
            pl.BlockSpec((_HID, _OUT), lambda i: (0, 0)),
            pl.BlockSpec((1, _OUT), lambda i: (0, 0)),
        ],
        out_specs=[
            pl.BlockSpec((_NC, _BN, 128), lambda i: (0, i, 0)),
            pl.BlockSpec((_BN, _OUT), lambda i: (i, 0)),
        ],
        out_shape=[
            jax.ShapeDtypeStruct((_NC, _N, 128), jnp.float32),
            jax.ShapeDtypeStruct((_N, _OUT), jnp.float32),
        ],
    )(agg1, x, deg, W1_l, W1_r, b1, W2_l, W2_r, b2)


def _tc2(agg2, deg, r):
    grid = (_N // _BN,)
    return pl.pallas_call(
        _tc2_body,
        grid=grid,
        in_specs=[
            pl.BlockSpec((_NC, _BN, 128), lambda i: (0, i, 0)),
            pl.BlockSpec((_NC, _BN, 128), lambda i: (0, i, 0)),
            pl.BlockSpec((_BN, _OUT), lambda i: (i, 0)),
        ],
        out_specs=pl.BlockSpec((_BN, _OUT), lambda i: (i, 0)),
        out_shape=jax.ShapeDtypeStruct((_N, _OUT), jnp.float32),
    )(agg2, deg, r)


def kernel(x, edge_index, W1_l, W1_r, b1, W2_l, W2_r, b2):
    src = edge_index[0]
    dst = edge_index[1]
    pad = _EPAD - _E
    iota = jnp.arange(pad, dtype=jnp.int32)
    src_p = jnp.concatenate([src, iota * 37 % _N])
    dst_p = jnp.concatenate([dst, _N + iota % (_ACC_ROWS - _N)])
    src2 = jnp.stack([src_p, src_p + _N]).reshape(_NC, _NCHUNKS, _CHUNK)
    dst2 = dst_p.reshape(_NCHUNKS, _CHUNK)
    x2 = jnp.concatenate([x[:, :128], x[:, 128:]], axis=0)

    agg1, deg = _make_seg(True)(x2, src2, dst2)
    p2, r = _tc1(agg1, x, deg, W1_l, W1_r, b1.reshape(1, _HID),
                 W2_l, W2_r, b2.reshape(1, _OUT))
    agg2, = _make_seg(False)(p2.reshape(_NC * _N, 128), src2, dst2)
    return _tc2(agg2, deg, r)


# speedup vs baseline: 1.0905x; 1.0815x over previous
import functools

import jax
import jax.numpy as jnp
from jax import lax
from jax.experimental import pallas as pl
from jax.experimental.pallas import tpu as pltpu
from jax.experimental.pallas import tpu_sc as plsc

_N = 10000
_E = 160000
_IN = 256
_HID = 512
_OUT = 256

_NC = 2
_NS = 16
_CHUNK = 128
_EPAD = 163840
_NCHUNKS = _EPAD // _CHUNK
_CPT = _NCHUNKS // _NS
_ZROWS = 632
_ACC_ROWS = _NS * _ZROWS


def _fill_rows(ref, nrows, val):
    v = jnp.full((16,), val, jnp.float32)
    ncol = ref.shape[1]

    def body(i, carry):
        for j in range(ncol // 16):
            ref[i, pl.ds(j * 16, 16)] = v
        return carry

    lax.fori_loop(0, nrows, body, 0)


def _zero_slice(acc, zbuf, zbase):
    for k in range(_ZROWS // _CHUNK):
        pltpu.sync_copy(zbuf, acc.at[pl.ds(zbase + k * _CHUNK, _CHUNK)])
    rem = _ZROWS % _CHUNK
    if rem:
        pltpu.sync_copy(zbuf.at[pl.ds(0, rem)],
                        acc.at[pl.ds(zbase + (_ZROWS // _CHUNK) * _CHUNK, rem)])


_W = 16
_NWIN = _CPT // _W
_QC = 64


def _seg_body(compute_deg, table, src2, dst2, *rest):
    if compute_deg:
        (out_agg, out_deg, src_w, dst_w, buf0, buf1, acc, sem0, sem1) = rest
    else:
        (out_agg, src_w, dst_w, buf0, buf1, acc, sem0, sem1) = rest
    c = lax.axis_index("c")
    s = lax.axis_index("s")
    zbase = s * _ZROWS

    bufs = (buf0, buf1)
    sems = (sem0, sem1)

    _fill_rows(buf0, _CHUNK, 0.0)
    _zero_slice(acc, buf0, zbase)

    if compute_deg:
        _fill_rows(buf1, _CHUNK, 1.0)
        plsc.subcore_barrier()

        def dwin(w, carry):
            base = c * (_NS * _DEG_CPT) + s * _DEG_CPT + w * _W
            pltpu.sync_copy(dst2.at[pl.ds(base, _W)], dst_w)
            for j in range(_W):
                pltpu.sync_copy(buf1, acc.at[dst_w.at[j]], add=True)
            return carry

        lax.fori_loop(0, _DEG_CPT // _W, dwin, 0)

        plsc.subcore_barrier()
        pltpu.sync_copy(acc.at[pl.ds(zbase, _ZROWS)],
                        out_deg.at[c, pl.ds(zbase, _ZROWS)])
        _zero_slice(acc, buf0, zbase)

    plsc.subcore_barrier()

    def window(w, carry):
        base = s * _CPT + w * _W
        pltpu.sync_copy(src2.at[c, pl.ds(base, _W)], src_w)
        pltpu.sync_copy(dst2.at[pl.ds(base, _W)], dst_w)
        cps = [None, None]
        cps[0] = pltpu.async_copy(table.at[src_w.at[0]], buf0, sem0)
        for j in range(_W):
            nxt = (j + 1) % 2
            if j + 1 < _W:
                cps[nxt] = pltpu.async_copy(table.at[src_w.at[j + 1]],
                                            bufs[nxt], sems[nxt])
            cps[j % 2].wait()
            pltpu.sync_copy(bufs[j % 2], acc.at[dst_w.at[j]], add=True)
        return carry

    lax.fori_loop(0, _NWIN, window, 0)

    plsc.subcore_barrier()

    pltpu.sync_copy(acc.at[pl.ds(zbase, _ZROWS)],
                    out_agg.at[c, pl.ds(zbase, _ZROWS)])


_DEG_CPT = _NCHUNKS // (_NC * _NS)


def _make_seg(compute_deg):
    mesh = plsc.VectorSubcoreMesh(core_axis_name="c", subcore_axis_name="s",
                                  num_cores=_NC, num_subcores=_NS)
    out_type = [jax.ShapeDtypeStruct((_NC, _ACC_ROWS, 128), jnp.float32)]
    if compute_deg:
        out_type.append(
            jax.ShapeDtypeStruct((_NC, _ACC_ROWS, 128), jnp.float32))
    scratch = [
        pltpu.VMEM((_W, _CHUNK), jnp.int32),
        pltpu.VMEM((_W, _CHUNK), jnp.int32),
        pltpu.VMEM((_CHUNK, 128), jnp.float32),
        pltpu.VMEM((_CHUNK, 128), jnp.float32),
        pltpu.VMEM_SHARED((_ACC_ROWS, 128), jnp.float32),
        pltpu.SemaphoreType.DMA,
        pltpu.SemaphoreType.DMA,
    ]
    return pl.kernel(
        functools.partial(_seg_body, compute_deg),
        out_type=tuple(out_type), mesh=mesh, scratch_types=scratch)


_BN = 400


def _tc1_body(agg_ref, x_ref, deg_ref, w1l_ref, w1r_ref, b1_ref,
              w2l_ref, w2r_ref, b2_ref, p4_ref, r_ref):
    deg = deg_ref[0, :, 0:1] + deg_ref[1, :, 0:1]
    inv = 1.0 / jnp.maximum(deg, 1.0)
    ssum = jnp.dot(agg_ref[0], w1l_ref[:128, :],
                   preferred_element_type=jnp.float32)
    ssum = ssum + jnp.dot(agg_ref[1], w1l_ref[128:, :],
                          preferred_element_type=jnp.float32)
    h = ssum * inv + jnp.dot(x_ref[...], w1r_ref[...],
                             preferred_element_type=jnp.float32) + b1_ref[...]
    h = jnp.maximum(h, 0.0)
    p = jnp.dot(h, w2l_ref[...], preferred_element_type=jnp.float32)
    p4_ref[0] = p[:, :128]
    p4_ref[1] = p[:, 128:]
    r_ref[...] = jnp.dot(h, w2r_ref[...],
                         preferred_element_type=jnp.float32) + b2_ref[...]


def _tc2_body(agg_ref, deg_ref, r_ref, out_ref):
    deg = deg_ref[0, :, 0:1] + deg_ref[1, :, 0:1]
    inv = 1.0 / jnp.maximum(deg, 1.0)
    out_ref[...] = jnp.concatenate(
        [agg_ref[0], agg_ref[1]], axis=1) * inv + r_ref[...]


def _tc1(agg1, x, deg, W1_l, W1_r, b1, W2_l, W2_r, b2):
    grid = (_N // _BN,)
    return pl.pallas_call(
        _tc1_body,
        grid=grid,
        in_specs=[
            pl.BlockSpec((_NC, _BN, 128), lambda i: (0, i, 0)),
            pl.BlockSpec((_BN, _IN), lambda i: (i, 0)),
            pl.BlockSpec((_NC, _BN, 128), lambda i: (0, i, 0)),
            pl.BlockSpec((_IN, _HID), lambda i: (0, 0)),
            pl.BlockSpec((_IN, _HID), lambda i: (0, 0)),
            pl.BlockSpec((1, _HID), lambda i: (0, 0)),
            pl.BlockSpec((_HID, _OUT), lambda i: (0, 0)),
            pl.BlockSpec((_HID, _OUT), lambda i: (0, 0)),
            pl.BlockSpec((1, _OUT), lambda i: (0, 0)),
        ],
        out_specs=[
            pl.BlockSpec((_NC, _BN, 128), lambda i: (0, i, 0)),
            pl.BlockSpec((_BN, _OUT), lambda i: (i, 0)),
        ],
        out_shape=[
            jax.ShapeDtypeStruct((_NC, _N, 128), jnp.float32),
            jax.ShapeDtypeStruct((_N, _OUT), jnp.float32),
        ],
    )(agg1, x, deg, W1_l, W1_r, b1, W2_l, W2_r, b2)


def _tc2(agg2, deg, r):
    grid = (_N // _BN,)
    return pl.pallas_call(
        _tc2_body,
        grid=grid,
        in_specs=[
            pl.BlockSpec((_NC, _BN, 128), lambda i: (0, i, 0)),
            pl.BlockSpec((_NC, _BN, 128), lambda i: (0, i, 0)),
            pl.BlockSpec((_BN, _OUT), lambda i: (i, 0)),
        ],
        out_specs=pl.BlockSpec((_BN, _OUT), lambda i: (i, 0)),
        out_shape=jax.ShapeDtypeStruct((_N, _OUT), jnp.float32),
    )(agg2, deg, r)


def kernel(x, edge_index, W1_l, W1_r, b1, W2_l, W2_r, b2):
    src = edge_index[0]
    dst = edge_index[1]
    pad = _EPAD - _E
    iota = jnp.arange(pad, dtype=jnp.int32)
    src_p = jnp.concatenate([src, iota * 37 % _N])
    dst_p = jnp.concatenate([dst, _N + iota % (_ACC_ROWS - _N)])
    src2 = jnp.stack([src_p, src_p + _N]).reshape(_NC, _NCHUNKS, _CHUNK)
    dst2 = dst_p.reshape(_NCHUNKS, _CHUNK)
    x2 = jnp.concatenate([x[:, :128], x[:, 128:]], axis=0)

    agg1, deg = _make_seg(True)(x2, src2, dst2)
    p2, r = _tc1(agg1, x, deg, W1_l, W1_r, b1.reshape(1, _HID),
                 W2_l, W2_r, b2.reshape(1, _OUT))
    agg2, = _make_seg(False)(p2.reshape(_NC * _N, 128), src2, dst2)
    return _tc2(agg2, deg, r)
